# baseline (speedup 1.0000x reference)
import functools

import jax
import jax.numpy as jnp
from jax import lax
from jax.experimental import pallas as pl
from jax.experimental.pallas import tpu as pltpu
from jax.experimental.pallas import tpu_sc as plsc

N = 10000
D = 128
E = 320000
NC = 2
NS = 16
NW = NC * NS
K = 128
NBUF = 3
STEPS = 81
EPAD = NW * STEPS * K
ACC_ROWS = 10112
STRIPE = ACC_ROWS // NS
DPT = E // NW
BETA = 0.5
CVAL = 1.0

_sc_mesh = plsc.VectorSubcoreMesh(core_axis_name="c", subcore_axis_name="s")


def _deg_body(dst_hbm, out_hbm, idx_v, deg_v):
    c = lax.axis_index("c")
    s = lax.axis_index("s")
    w = s * NC + c
    pltpu.sync_copy(dst_hbm.at[w], idx_v)
    zeros16 = jnp.zeros((16,), jnp.float32)

    def zstep(i, _):
        deg_v[pl.ds(i * 16, 16)] = zeros16
        return 0

    lax.fori_loop(0, DPT // 16, zstep, 0)
    ones16 = jnp.ones((16,), jnp.float32)

    def astep(i, _):
        idx = idx_v[pl.ds(i * 16, 16)]
        plsc.addupdate_scatter(deg_v, [idx], ones16)
        return 0

    lax.fori_loop(0, DPT // 16, astep, 0)
    pltpu.sync_copy(deg_v, out_hbm.at[w])


_deg_call = functools.partial(
    pl.kernel,
    out_type=jax.ShapeDtypeStruct((NW, DPT), jnp.float32),
    mesh=_sc_mesh,
    scratch_types=[
        pltpu.VMEM((DPT,), jnp.int32),
        pltpu.VMEM((DPT,), jnp.float32),
    ],
    compiler_params=pltpu.CompilerParams(needs_layout_passes=False),
)(_deg_body)


def _scatter_body(g_hbm, srcp_hbm, dstp_hbm, out_hbm,
                  idxs, idxd, bufs, isems, gsems, ssems, acc):
    c = lax.axis_index("c")
    s = lax.axis_index("s")
    w = s * NC + c
    zeros16 = jnp.zeros((16,), jnp.float32)

    def zstep(r, _):
        for j in range(D // 16):
            bufs[0][r, pl.ds(j * 16, 16)] = zeros16
        return 0

    lax.fori_loop(0, K, zstep, 0)
    for off in (0, K, 2 * K, 3 * K, STRIPE - K):
        pltpu.sync_copy(bufs[0], acc.at[pl.ds(s * STRIPE + off, K)])
    for b in range(NBUF):
        pltpu.async_copy(srcp_hbm.at[w, b], idxs[b], isems[b])
        pltpu.async_copy(dstp_hbm.at[w, b], idxd[b], isems[b])
    for b in range(NBUF):
        pltpu.make_async_copy(srcp_hbm.at[w, b], idxs[b], isems[b]).wait()
        pltpu.make_async_copy(dstp_hbm.at[w, b], idxd[b], isems[b]).wait()
        pltpu.async_copy(g_hbm.at[idxs[b]], bufs[b], gsems[b])
    plsc.subcore_barrier()

    H = STEPS // NBUF

    def step(j, _):
        base = j * NBUF
        for b in range(NBUF):
            pltpu.make_async_copy(g_hbm.at[idxs[b]], bufs[b], gsems[b]).wait()
            pltpu.async_copy(bufs[b], acc.at[idxd[b]], ssems[b], add=True)

        @pl.when(j < H - 1)
        def _():
            for b in range(NBUF):
                i = base + b + NBUF
                pltpu.make_async_copy(bufs[b], acc.at[idxd[b]], ssems[b]).wait()
                pltpu.async_copy(srcp_hbm.at[w, i], idxs[b], isems[b])
                pltpu.async_copy(dstp_hbm.at[w, i], idxd[b], isems[b])
                pltpu.make_async_copy(srcp_hbm.at[w, i], idxs[b], isems[b]).wait()
                pltpu.make_async_copy(dstp_hbm.at[w, i], idxd[b], isems[b]).wait()
                pltpu.async_copy(g_hbm.at[idxs[b]], bufs[b], gsems[b])

        return 0

    lax.fori_loop(0, H, step, 0)
    for b in range(NBUF):
        pltpu.make_async_copy(bufs[b], acc.at[idxd[b]], ssems[b]).wait()
    plsc.subcore_barrier()
    pltpu.sync_copy(acc.at[pl.ds(s * STRIPE, STRIPE)],
                    out_hbm.at[c, pl.ds(s * STRIPE, STRIPE)])


_scatter_call = functools.partial(
    pl.kernel,
    out_type=jax.ShapeDtypeStruct((NC, ACC_ROWS, D), jnp.float32),
    mesh=_sc_mesh,
    scratch_types=[
        [pltpu.VMEM((K,), jnp.int32)] * NBUF,
        [pltpu.VMEM((K,), jnp.int32)] * NBUF,
        [pltpu.VMEM((K, D), jnp.float32)] * NBUF,
        [pltpu.SemaphoreType.DMA] * NBUF,
        [pltpu.SemaphoreType.DMA] * NBUF,
        [pltpu.SemaphoreType.DMA] * NBUF,
        pltpu.VMEM_SHARED((ACC_ROWS, D), jnp.float32),
    ],
    compiler_params=pltpu.CompilerParams(needs_layout_passes=False),
)(_scatter_body)


R = 2000
G = N // R


def _dis_body(degp_ref, dis_ref):
    deg = jnp.sum(degp_ref[...], axis=0) + 1.0
    dis_ref[...] = lax.rsqrt(deg)[:, None]


_dis_call = pl.pallas_call(
    _dis_body,
    out_shape=jax.ShapeDtypeStruct((N, 1), jnp.float32),
)


def _tc1_body(x_ref, Win_ref, bin_ref, Wg_ref, dis_ref, h_ref, g_ref):
    h = jnp.dot(x_ref[...], Win_ref[...], preferred_element_type=jnp.float32) + bin_ref[...]
    g = jnp.dot(h, Wg_ref[...], preferred_element_type=jnp.float32) * dis_ref[...]
    h_ref[...] = h
    g_ref[...] = g


_tc1_call = pl.pallas_call(
    _tc1_body,
    grid=(G,),
    in_specs=[
        pl.BlockSpec((R, D), lambda b: (b, 0)),
        pl.BlockSpec((D, D), lambda b: (0, 0)),
        pl.BlockSpec((1, D), lambda b: (0, 0)),
        pl.BlockSpec((D, D), lambda b: (0, 0)),
        pl.BlockSpec((R, 1), lambda b: (b, 0)),
    ],
    out_specs=[
        pl.BlockSpec((R, D), lambda b: (b, 0)),
        pl.BlockSpec((R, D), lambda b: (b, 0)),
    ],
    out_shape=[
        jax.ShapeDtypeStruct((N, D), jnp.float32),
        jax.ShapeDtypeStruct((N, D), jnp.float32),
    ],
)


def _mix(parts, g, h, dis, bg, lnw, lnb):
    acc = parts[0] + parts[1]
    y = dis * (acc + g) + bg
    mu = jnp.mean(y, axis=-1, keepdims=True)
    var = jnp.mean((y - mu) ** 2, axis=-1, keepdims=True)
    z = (y - mu) * lax.rsqrt(var + 1e-5) * lnw + lnb
    tilde = (CVAL - BETA) * jnp.maximum(z, 0.0) + BETA * y
    return tilde + (CVAL - BETA) * h


def _tc2_body(parts_ref, g_ref, h_ref, dis_ref, bg_ref, lnw_ref, lnb_ref,
              Wg_ref, h1_ref, g1_ref):
    dis = dis_ref[...]
    h1 = _mix(parts_ref[...], g_ref[...], h_ref[...], dis,
              bg_ref[...], lnw_ref[...], lnb_ref[...])
    h1_ref[...] = h1
    g1_ref[...] = jnp.dot(h1, Wg_ref[...], preferred_element_type=jnp.float32) * dis


_tc2_call = pl.pallas_call(
    _tc2_body,
    grid=(G,),
    in_specs=[
        pl.BlockSpec((2, R, D), lambda b: (0, b, 0)),
        pl.BlockSpec((R, D), lambda b: (b, 0)),
        pl.BlockSpec((R, D), lambda b: (b, 0)),
        pl.BlockSpec((R, 1), lambda b: (b, 0)),
        pl.BlockSpec((1, D), lambda b: (0, 0)),
        pl.BlockSpec((1, D), lambda b: (0, 0)),
        pl.BlockSpec((1, D), lambda b: (0, 0)),
        pl.BlockSpec((D, D), lambda b: (0, 0)),
    ],
    out_specs=[
        pl.BlockSpec((R, D), lambda b: (b, 0)),
        pl.BlockSpec((R, D), lambda b: (b, 0)),
    ],
    out_shape=[
        jax.ShapeDtypeStruct((N, D), jnp.float32),
        jax.ShapeDtypeStruct((N, D), jnp.float32),
    ],
)


def _tc3_body(parts_ref, g_ref, h_ref, dis_ref, bg_ref, lnw_ref, lnb_ref,
              Wout_ref, bout_ref, out_ref):
    h2 = _mix(parts_ref[...], g_ref[...], h_ref[...], dis_ref[...],
              bg_ref[...], lnw_ref[...], lnb_ref[...])
    out_ref[...] = (jnp.dot(h2, Wout_ref[...], preferred_element_type=jnp.float32)
                    + bout_ref[...])


_tc3_call = pl.pallas_call(
    _tc3_body,
    grid=(G,),
    in_specs=[
        pl.BlockSpec((2, R, D), lambda b: (0, b, 0)),
        pl.BlockSpec((R, D), lambda b: (b, 0)),
        pl.BlockSpec((R, D), lambda b: (b, 0)),
        pl.BlockSpec((R, 1), lambda b: (b, 0)),
        pl.BlockSpec((1, D), lambda b: (0, 0)),
        pl.BlockSpec((1, D), lambda b: (0, 0)),
        pl.BlockSpec((1, D), lambda b: (0, 0)),
        pl.BlockSpec((D, D), lambda b: (0, 0)),
        pl.BlockSpec((1, D), lambda b: (0, 0)),
    ],
    out_specs=pl.BlockSpec((R, D), lambda b: (b, 0)),
    out_shape=jax.ShapeDtypeStruct((N, D), jnp.float32),
)


def kernel(x, edge_index, W_in, b_in, Wg0, bg0, lnw0, lnb0, Wg1, bg1, lnw1,
           lnb1, W_out, b_out):
    src = edge_index[0]
    dst = edge_index[1]
    pad = EPAD - E
    srcp = jnp.concatenate([src, jnp.zeros((pad,), src.dtype)]).reshape(NW, STEPS, K)
    dstp = jnp.concatenate([dst, jnp.full((pad,), N, dst.dtype)]).reshape(NW, STEPS, K)
    dst2d = dst.reshape(NW, DPT)

    vec = lambda v: v.reshape(1, D)

    degp = _deg_call(dst2d)
    dis = _dis_call(degp)
    h, g0 = _tc1_call(x, W_in, vec(b_in), Wg0, dis)
    p0 = _scatter_call(g0, srcp, dstp)
    h1, g1 = _tc2_call(p0, g0, h, dis, vec(bg0), vec(lnw0), vec(lnb0), Wg1)
    p1 = _scatter_call(g1, srcp, dstp)
    out = _tc3_call(p1, g1, h1, dis, vec(bg1), vec(lnw1), vec(lnb1), W_out,
                    vec(b_out))
    return out


# speedup vs baseline: 1.1159x; 1.1159x over previous
import functools

import jax
import jax.numpy as jnp
from jax import lax
from jax.experimental import pallas as pl
from jax.experimental.pallas import tpu as pltpu
from jax.experimental.pallas import tpu_sc as plsc

N = 10000
D = 128
E = 320000
NC = 2
NS = 16
NW = NC * NS
K = 256
STEPS = 40
EPAD = NW * STEPS * K
ACC_ROWS = 10112
STRIPE = ACC_ROWS // NS
DPT = E // NW
BETA = 0.5
CVAL = 1.0

_sc_mesh = plsc.VectorSubcoreMesh(core_axis_name="c", subcore_axis_name="s")


def _deg_body(dst_hbm, out_hbm, idx_v, deg_v):
    c = lax.axis_index("c")
    s = lax.axis_index("s")
    w = s * NC + c
    pltpu.sync_copy(dst_hbm.at[w], idx_v)
    zeros16 = jnp.zeros((16,), jnp.float32)

    def zstep(i, _):
        deg_v[pl.ds(i * 16, 16)] = zeros16
        return 0

    lax.fori_loop(0, DPT // 16, zstep, 0)
    ones16 = jnp.ones((16,), jnp.float32)

    def astep(i, _):
        idx = idx_v[pl.ds(i * 16, 16)]
        plsc.addupdate_scatter(deg_v, [idx], ones16)
        return 0

    lax.fori_loop(0, DPT // 16, astep, 0)
    pltpu.sync_copy(deg_v, out_hbm.at[w])


_deg_call = functools.partial(
    pl.kernel,
    out_type=jax.ShapeDtypeStruct((NW, DPT), jnp.float32),
    mesh=_sc_mesh,
    scratch_types=[
        pltpu.VMEM((DPT,), jnp.int32),
        pltpu.VMEM((DPT,), jnp.float32),
    ],
    compiler_params=pltpu.CompilerParams(needs_layout_passes=False),
)(_deg_body)


def _scatter_body(g_hbm, srcp_hbm, dstp_hbm, out_hbm,
                  idxs, idxd, buf, ssem, acc):
    c = lax.axis_index("c")
    s = lax.axis_index("s")
    w = s * NC + c
    zeros16 = jnp.zeros((16,), jnp.float32)

    def zstep(r, _):
        for j in range(D // 16):
            buf[r, pl.ds(j * 16, 16)] = zeros16
        return 0

    lax.fori_loop(0, K, zstep, 0)
    off = 0
    while off < STRIPE:
        n = min(K, STRIPE - off)
        pltpu.sync_copy(buf.at[pl.ds(0, n)],
                        acc.at[pl.ds(s * STRIPE + off, n)])
        off += n
    plsc.subcore_barrier()

    def step(i, _):
        pltpu.sync_copy(srcp_hbm.at[w, i], idxs)
        pltpu.sync_copy(dstp_hbm.at[w, i], idxd)
        pltpu.sync_copy(g_hbm.at[idxs], buf)
        pltpu.async_copy(buf, acc.at[idxd], ssem, add=True)
        pltpu.make_async_copy(buf, acc.at[idxd], ssem).wait()
        return 0

    lax.fori_loop(0, STEPS, step, 0)
    plsc.subcore_barrier()
    pltpu.sync_copy(acc.at[pl.ds(s * STRIPE, STRIPE)],
                    out_hbm.at[c, pl.ds(s * STRIPE, STRIPE)])


_scatter_call = functools.partial(
    pl.kernel,
    out_type=jax.ShapeDtypeStruct((NC, ACC_ROWS, D), jnp.float32),
    mesh=_sc_mesh,
    scratch_types=[
        pltpu.VMEM((K,), jnp.int32),
        pltpu.VMEM((K,), jnp.int32),
        pltpu.VMEM((K, D), jnp.float32),
        pltpu.SemaphoreType.DMA,
        pltpu.VMEM_SHARED((ACC_ROWS, D), jnp.float32),
    ],
    compiler_params=pltpu.CompilerParams(needs_layout_passes=False),
)(_scatter_body)


R = 2000
G = N // R


def _dis_body(degp_ref, dis_ref):
    deg = jnp.sum(degp_ref[...], axis=0) + 1.0
    dis_ref[...] = lax.rsqrt(deg)[:, None]


_dis_call = pl.pallas_call(
    _dis_body,
    out_shape=jax.ShapeDtypeStruct((N, 1), jnp.float32),
)


def _tc1_body(x_ref, Win_ref, bin_ref, Wg_ref, dis_ref, h_ref, g_ref):
    h = jnp.dot(x_ref[...], Win_ref[...], preferred_element_type=jnp.float32) + bin_ref[...]
    g = jnp.dot(h, Wg_ref[...], preferred_element_type=jnp.float32) * dis_ref[...]
    h_ref[...] = h
    g_ref[...] = g


_tc1_call = pl.pallas_call(
    _tc1_body,
    grid=(G,),
    in_specs=[
        pl.BlockSpec((R, D), lambda b: (b, 0)),
        pl.BlockSpec((D, D), lambda b: (0, 0)),
        pl.BlockSpec((1, D), lambda b: (0, 0)),
        pl.BlockSpec((D, D), lambda b: (0, 0)),
        pl.BlockSpec((R, 1), lambda b: (b, 0)),
    ],
    out_specs=[
        pl.BlockSpec((R, D), lambda b: (b, 0)),
        pl.BlockSpec((R, D), lambda b: (b, 0)),
    ],
    out_shape=[
        jax.ShapeDtypeStruct((N, D), jnp.float32),
        jax.ShapeDtypeStruct((N, D), jnp.float32),
    ],
)


def _mix(parts, g, h, dis, bg, lnw, lnb):
    acc = parts[0] + parts[1]
    y = dis * (acc + g) + bg
    mu = jnp.mean(y, axis=-1, keepdims=True)
    var = jnp.mean((y - mu) ** 2, axis=-1, keepdims=True)
    z = (y - mu) * lax.rsqrt(var + 1e-5) * lnw + lnb
    tilde = (CVAL - BETA) * jnp.maximum(z, 0.0) + BETA * y
    return tilde + (CVAL - BETA) * h


def _tc2_body(parts_ref, g_ref, h_ref, dis_ref, bg_ref, lnw_ref, lnb_ref,
              Wg_ref, h1_ref, g1_ref):
    dis = dis_ref[...]
    h1 = _mix(parts_ref[...], g_ref[...], h_ref[...], dis,
              bg_ref[...], lnw_ref[...], lnb_ref[...])
    h1_ref[...] = h1
    g1_ref[...] = jnp.dot(h1, Wg_ref[...], preferred_element_type=jnp.float32) * dis


_tc2_call = pl.pallas_call(
    _tc2_body,
    grid=(G,),
    in_specs=[
        pl.BlockSpec((2, R, D), lambda b: (0, b, 0)),
        pl.BlockSpec((R, D), lambda b: (b, 0)),
        pl.BlockSpec((R, D), lambda b: (b, 0)),
        pl.BlockSpec((R, 1), lambda b: (b, 0)),
        pl.BlockSpec((1, D), lambda b: (0, 0)),
        pl.BlockSpec((1, D), lambda b: (0, 0)),
        pl.BlockSpec((1, D), lambda b: (0, 0)),
        pl.BlockSpec((D, D), lambda b: (0, 0)),
    ],
    out_specs=[
        pl.BlockSpec((R, D), lambda b: (b, 0)),
        pl.BlockSpec((R, D), lambda b: (b, 0)),
    ],
    out_shape=[
        jax.ShapeDtypeStruct((N, D), jnp.float32),
        jax.ShapeDtypeStruct((N, D), jnp.float32),
    ],
)


def _tc3_body(parts_ref, g_ref, h_ref, dis_ref, bg_ref, lnw_ref, lnb_ref,
              Wout_ref, bout_ref, out_ref):
    h2 = _mix(parts_ref[...], g_ref[...], h_ref[...], dis_ref[...],
              bg_ref[...], lnw_ref[...], lnb_ref[...])
    out_ref[...] = (jnp.dot(h2, Wout_ref[...], preferred_element_type=jnp.float32)
                    + bout_ref[...])


_tc3_call = pl.pallas_call(
    _tc3_body,
    grid=(G,),
    in_specs=[
        pl.BlockSpec((2, R, D), lambda b: (0, b, 0)),
        pl.BlockSpec((R, D), lambda b: (b, 0)),
        pl.BlockSpec((R, D), lambda b: (b, 0)),
        pl.BlockSpec((R, 1), lambda b: (b, 0)),
        pl.BlockSpec((1, D), lambda b: (0, 0)),
        pl.BlockSpec((1, D), lambda b: (0, 0)),
        pl.BlockSpec((1, D), lambda b: (0, 0)),
        pl.BlockSpec((D, D), lambda b: (0, 0)),
        pl.BlockSpec((1, D), lambda b: (0, 0)),
    ],
    out_specs=pl.BlockSpec((R, D), lambda b: (b, 0)),
    out_shape=jax.ShapeDtypeStruct((N, D), jnp.float32),
)


def kernel(x, edge_index, W_in, b_in, Wg0, bg0, lnw0, lnb0, Wg1, bg1, lnw1,
           lnb1, W_out, b_out):
    src = edge_index[0]
    dst = edge_index[1]
    pad = EPAD - E
    srcp = jnp.concatenate([src, jnp.zeros((pad,), src.dtype)]).reshape(NW, STEPS, K)
    dstp = jnp.concatenate([dst, jnp.full((pad,), N, dst.dtype)]).reshape(NW, STEPS, K)
    dst2d = dst.reshape(NW, DPT)

    vec = lambda v: v.reshape(1, D)

    degp = _deg_call(dst2d)
    dis = _dis_call(degp)
    h, g0 = _tc1_call(x, W_in, vec(b_in), Wg0, dis)
    p0 = _scatter_call(g0, srcp, dstp)
    h1, g1 = _tc2_call(p0, g0, h, dis, vec(bg0), vec(lnw0), vec(lnb0), Wg1)
    p1 = _scatter_call(g1, srcp, dstp)
    out = _tc3_call(p1, g1, h1, dis, vec(bg1), vec(lnw1), vec(lnb1), W_out,
                    vec(b_out))
    return out


# speedup vs baseline: 3.0225x; 2.7085x over previous
import functools

import jax
import jax.numpy as jnp
from jax import lax
from jax.experimental import pallas as pl
from jax.experimental.pallas import tpu as pltpu
from jax.experimental.pallas import tpu_sc as plsc

N = 10000
D = 128
E = 320000
NC = 2
NS = 16
NW = NC * NS
K = 256
STEPS = 40
EPAD = NW * STEPS * K
ACC_ROWS = 10112
STRIPE = ACC_ROWS // NS
DPT = E // NW
BETA = 0.5
CVAL = 1.0

_sc_mesh = plsc.VectorSubcoreMesh(core_axis_name="c", subcore_axis_name="s")


def _deg_body(dst_hbm, out_hbm, idx_v, deg_v):
    c = lax.axis_index("c")
    s = lax.axis_index("s")
    w = s * NC + c
    pltpu.sync_copy(dst_hbm.at[w], idx_v)
    zeros16 = jnp.zeros((16,), jnp.float32)

    def zstep(i, _):
        deg_v[pl.ds(i * 16, 16)] = zeros16
        return 0

    lax.fori_loop(0, DPT // 16, zstep, 0)
    ones16 = jnp.ones((16,), jnp.float32)

    def astep(i, _):
        idx = idx_v[pl.ds(i * 16, 16)]
        plsc.addupdate_scatter(deg_v, [idx], ones16)
        return 0

    lax.fori_loop(0, DPT // 16, astep, 0)
    pltpu.sync_copy(deg_v, out_hbm.at[w])


_deg_call = functools.partial(
    pl.kernel,
    out_type=jax.ShapeDtypeStruct((NW, DPT), jnp.float32),
    mesh=_sc_mesh,
    scratch_types=[
        pltpu.VMEM((DPT,), jnp.int32),
        pltpu.VMEM((DPT,), jnp.float32),
    ],
    compiler_params=pltpu.CompilerParams(needs_layout_passes=False),
)(_deg_body)


def _scatter_body(g_hbm, srcp_hbm, dstp_hbm, out_hbm,
                  idxs, idxd, buf, ssem, acc):
    c = lax.axis_index("c")
    s = lax.axis_index("s")
    w = s * NC + c
    zeros16 = jnp.zeros((16,), jnp.float32)

    def zstep(r, _):
        for j in range(D // 16):
            buf[r, pl.ds(j * 16, 16)] = zeros16
        return 0

    lax.fori_loop(0, K, zstep, 0)
    off = 0
    while off < STRIPE:
        n = min(K, STRIPE - off)
        pltpu.sync_copy(buf.at[pl.ds(0, n)],
                        acc.at[pl.ds(s * STRIPE + off, n)])
        off += n
    plsc.subcore_barrier()

    def step(i, _):
        pltpu.sync_copy(srcp_hbm.at[w, i], idxs)
        pltpu.sync_copy(dstp_hbm.at[w, i], idxd)
        pltpu.sync_copy(g_hbm.at[idxs], buf)
        pltpu.async_copy(buf, acc.at[idxd], ssem, add=True)
        pltpu.make_async_copy(buf, acc.at[idxd], ssem).wait()
        return 0

    lax.fori_loop(0, STEPS, step, 0)
    plsc.subcore_barrier()
    pltpu.sync_copy(acc.at[pl.ds(s * STRIPE, STRIPE)],
                    out_hbm.at[c, pl.ds(s * STRIPE, STRIPE)])


_scatter_call = functools.partial(
    pl.kernel,
    out_type=jax.ShapeDtypeStruct((NC, ACC_ROWS, D), jnp.float32),
    mesh=_sc_mesh,
    scratch_types=[
        pltpu.VMEM((K,), jnp.int32),
        pltpu.VMEM((K,), jnp.int32),
        pltpu.VMEM((K, D), jnp.float32),
        pltpu.SemaphoreType.DMA,
        pltpu.VMEM_SHARED((ACC_ROWS, D), jnp.float32),
    ],
    compiler_params=pltpu.CompilerParams(needs_layout_passes=False),
)(_scatter_body)


R = 2000
G = N // R


def _dis_body(degp_ref, dis_ref):
    deg = jnp.sum(degp_ref[...], axis=0) + 1.0
    dis_ref[...] = lax.rsqrt(deg)[:, None]


_dis_call = pl.pallas_call(
    _dis_body,
    out_shape=jax.ShapeDtypeStruct((N, 1), jnp.float32),
)


def _tc1_body(x_ref, Win_ref, bin_ref, Wg_ref, dis_ref, h_ref, g_ref):
    h = jnp.dot(x_ref[...], Win_ref[...], preferred_element_type=jnp.float32) + bin_ref[...]
    g = jnp.dot(h, Wg_ref[...], preferred_element_type=jnp.float32) * dis_ref[...]
    h_ref[...] = h
    g_ref[...] = g


_tc1_call = pl.pallas_call(
    _tc1_body,
    grid=(G,),
    in_specs=[
        pl.BlockSpec((R, D), lambda b: (b, 0)),
        pl.BlockSpec((D, D), lambda b: (0, 0)),
        pl.BlockSpec((1, D), lambda b: (0, 0)),
        pl.BlockSpec((D, D), lambda b: (0, 0)),
        pl.BlockSpec((R, 1), lambda b: (b, 0)),
    ],
    out_specs=[
        pl.BlockSpec((R, D), lambda b: (b, 0)),
        pl.BlockSpec((R, D), lambda b: (b, 0)),
    ],
    out_shape=[
        jax.ShapeDtypeStruct((N, D), jnp.float32),
        jax.ShapeDtypeStruct((N, D), jnp.float32),
    ],
)


def _mix(parts, g, h, dis, bg, lnw, lnb):
    acc = parts[0] + parts[1]
    y = dis * (acc + g) + bg
    mu = jnp.mean(y, axis=-1, keepdims=True)
    var = jnp.mean((y - mu) ** 2, axis=-1, keepdims=True)
    z = (y - mu) * lax.rsqrt(var + 1e-5) * lnw + lnb
    tilde = (CVAL - BETA) * jnp.maximum(z, 0.0) + BETA * y
    return tilde + (CVAL - BETA) * h


def _tc2_body(parts_ref, g_ref, h_ref, dis_ref, bg_ref, lnw_ref, lnb_ref,
              Wg_ref, h1_ref, g1_ref):
    dis = dis_ref[...]
    h1 = _mix(parts_ref[...], g_ref[...], h_ref[...], dis,
              bg_ref[...], lnw_ref[...], lnb_ref[...])
    h1_ref[...] = h1
    g1_ref[...] = jnp.dot(h1, Wg_ref[...], preferred_element_type=jnp.float32) * dis


_tc2_call = pl.pallas_call(
    _tc2_body,
    grid=(G,),
    in_specs=[
        pl.BlockSpec((2, R, D), lambda b: (0, b, 0)),
        pl.BlockSpec((R, D), lambda b: (b, 0)),
        pl.BlockSpec((R, D), lambda b: (b, 0)),
        pl.BlockSpec((R, 1), lambda b: (b, 0)),
        pl.BlockSpec((1, D), lambda b: (0, 0)),
        pl.BlockSpec((1, D), lambda b: (0, 0)),
        pl.BlockSpec((1, D), lambda b: (0, 0)),
        pl.BlockSpec((D, D), lambda b: (0, 0)),
    ],
    out_specs=[
        pl.BlockSpec((R, D), lambda b: (b, 0)),
        pl.BlockSpec((R, D), lambda b: (b, 0)),
    ],
    out_shape=[
        jax.ShapeDtypeStruct((N, D), jnp.float32),
        jax.ShapeDtypeStruct((N, D), jnp.float32),
    ],
)


def _tc3_body(parts_ref, g_ref, h_ref, dis_ref, bg_ref, lnw_ref, lnb_ref,
              Wout_ref, bout_ref, out_ref):
    h2 = _mix(parts_ref[...], g_ref[...], h_ref[...], dis_ref[...],
              bg_ref[...], lnw_ref[...], lnb_ref[...])
    out_ref[...] = (jnp.dot(h2, Wout_ref[...], preferred_element_type=jnp.float32)
                    + bout_ref[...])


_tc3_call = pl.pallas_call(
    _tc3_body,
    grid=(G,),
    in_specs=[
        pl.BlockSpec((2, R, D), lambda b: (0, b, 0)),
        pl.BlockSpec((R, D), lambda b: (b, 0)),
        pl.BlockSpec((R, D), lambda b: (b, 0)),
        pl.BlockSpec((R, 1), lambda b: (b, 0)),
        pl.BlockSpec((1, D), lambda b: (0, 0)),
        pl.BlockSpec((1, D), lambda b: (0, 0)),
        pl.BlockSpec((1, D), lambda b: (0, 0)),
        pl.BlockSpec((D, D), lambda b: (0, 0)),
        pl.BlockSpec((1, D), lambda b: (0, 0)),
    ],
    out_specs=pl.BlockSpec((R, D), lambda b: (b, 0)),
    out_shape=jax.ShapeDtypeStruct((N, D), jnp.float32),
)


def kernel(x, edge_index, W_in, b_in, Wg0, bg0, lnw0, lnb0, Wg1, bg1, lnw1,
           lnb1, W_out, b_out):
    src = edge_index[0]
    dst = edge_index[1]
    pad = EPAD - E
    pad_src = jnp.arange(pad, dtype=src.dtype) % N
    pad_dst = N + jnp.arange(pad, dtype=dst.dtype) % (ACC_ROWS - N)
    srcp = jnp.concatenate([src, pad_src]).reshape(NW, STEPS, K)
    dstp = jnp.concatenate([dst, pad_dst]).reshape(NW, STEPS, K)
    dst2d = dst.reshape(NW, DPT)

    vec = lambda v: v.reshape(1, D)

    degp = _deg_call(dst2d)
    dis = _dis_call(degp)
    h, g0 = _tc1_call(x, W_in, vec(b_in), Wg0, dis)
    p0 = _scatter_call(g0, srcp, dstp)
    h1, g1 = _tc2_call(p0, g0, h, dis, vec(bg0), vec(lnw0), vec(lnb0), Wg1)
    p1 = _scatter_call(g1, srcp, dstp)
    out = _tc3_call(p1, g1, h1, dis, vec(bg1), vec(lnw1), vec(lnb1), W_out,
                    vec(b_out))
    return out


# speedup vs baseline: 3.6186x; 1.1972x over previous
import functools

import jax
import jax.numpy as jnp
from jax import lax
from jax.experimental import pallas as pl
from jax.experimental.pallas import tpu as pltpu
from jax.experimental.pallas import tpu_sc as plsc

N = 10000
D = 128
E = 320000
NC = 2
NS = 16
NW = NC * NS
K = 128
STEPS = 80
EPAD = NW * STEPS * K
ACC_ROWS = 10112
STRIPE = ACC_ROWS // NS
DPT = E // NW
BETA = 0.5
CVAL = 1.0

_sc_mesh = plsc.VectorSubcoreMesh(core_axis_name="c", subcore_axis_name="s")


def _deg_body(dst_hbm, out_hbm, idx_v, deg_v):
    c = lax.axis_index("c")
    s = lax.axis_index("s")
    w = s * NC + c
    pltpu.sync_copy(dst_hbm.at[w], idx_v)
    zeros16 = jnp.zeros((16,), jnp.float32)

    def zstep(i, _):
        deg_v[pl.ds(i * 16, 16)] = zeros16
        return 0

    lax.fori_loop(0, DPT // 16, zstep, 0)
    ones16 = jnp.ones((16,), jnp.float32)

    def astep(i, _):
        idx = idx_v[pl.ds(i * 16, 16)]
        plsc.addupdate_scatter(deg_v, [idx], ones16)
        return 0

    lax.fori_loop(0, DPT // 16, astep, 0)
    pltpu.sync_copy(deg_v, out_hbm.at[w])


_deg_call = functools.partial(
    pl.kernel,
    out_type=jax.ShapeDtypeStruct((NW, DPT), jnp.float32),
    mesh=_sc_mesh,
    scratch_types=[
        pltpu.VMEM((DPT,), jnp.int32),
        pltpu.VMEM((DPT,), jnp.float32),
    ],
    compiler_params=pltpu.CompilerParams(needs_layout_passes=False),
)(_deg_body)


def _scatter_body(g_hbm, srcp_hbm, dstp_hbm, out_hbm,
                  idxs, idxd, bufs, gsems, ssems, acc):
    c = lax.axis_index("c")
    s = lax.axis_index("s")
    w = s * NC + c
    zeros16 = jnp.zeros((16,), jnp.float32)

    def zstep(r, _):
        for j in range(D // 16):
            bufs[0][r, pl.ds(j * 16, 16)] = zeros16
        return 0

    lax.fori_loop(0, K, zstep, 0)
    off = 0
    while off < STRIPE:
        n = min(K, STRIPE - off)
        pltpu.sync_copy(bufs[0].at[pl.ds(0, n)],
                        acc.at[pl.ds(s * STRIPE + off, n)])
        off += n
    plsc.subcore_barrier()

    pltpu.sync_copy(srcp_hbm.at[w, 0], idxs[0])
    pltpu.sync_copy(dstp_hbm.at[w, 0], idxd[0])
    pltpu.async_copy(g_hbm.at[idxs[0]], bufs[0], gsems[0])

    H = STEPS // 2

    def _prep(slot, chunk):
        pltpu.make_async_copy(bufs[slot], acc.at[idxd[slot]], ssems[slot]).wait()
        pltpu.sync_copy(srcp_hbm.at[w, chunk], idxs[slot])
        pltpu.sync_copy(dstp_hbm.at[w, chunk], idxd[slot])
        pltpu.async_copy(g_hbm.at[idxs[slot]], bufs[slot], gsems[slot])

    def _fire(slot):
        pltpu.make_async_copy(g_hbm.at[idxs[slot]], bufs[slot], gsems[slot]).wait()
        pltpu.async_copy(bufs[slot], acc.at[idxd[slot]], ssems[slot], add=True)

    def step(j, _):
        @pl.when(j >= 1)
        def _():
            _prep(1, 2 * j + 1)

        @pl.when(j < 1)
        def _():
            pltpu.sync_copy(srcp_hbm.at[w, 1], idxs[1])
            pltpu.sync_copy(dstp_hbm.at[w, 1], idxd[1])
            pltpu.async_copy(g_hbm.at[idxs[1]], bufs[1], gsems[1])

        _fire(0)

        @pl.when(j < H - 1)
        def _():
            _prep(0, 2 * j + 2)

        _fire(1)
        return 0

    lax.fori_loop(0, H, step, 0)
    for b in (0, 1):
        pltpu.make_async_copy(bufs[b], acc.at[idxd[b]], ssems[b]).wait()
    plsc.subcore_barrier()
    pltpu.sync_copy(acc.at[pl.ds(s * STRIPE, STRIPE)],
                    out_hbm.at[c, pl.ds(s * STRIPE, STRIPE)])


_scatter_call = functools.partial(
    pl.kernel,
    out_type=jax.ShapeDtypeStruct((NC, ACC_ROWS, D), jnp.float32),
    mesh=_sc_mesh,
    scratch_types=[
        [pltpu.VMEM((K,), jnp.int32)] * 2,
        [pltpu.VMEM((K,), jnp.int32)] * 2,
        [pltpu.VMEM((K, D), jnp.float32)] * 2,
        [pltpu.SemaphoreType.DMA] * 2,
        [pltpu.SemaphoreType.DMA] * 2,
        pltpu.VMEM_SHARED((ACC_ROWS, D), jnp.float32),
    ],
    compiler_params=pltpu.CompilerParams(needs_layout_passes=False),
)(_scatter_body)


R = 2000
G = N // R


def _dis_body(degp_ref, dis_ref):
    deg = jnp.sum(degp_ref[...], axis=0) + 1.0
    dis_ref[...] = lax.rsqrt(deg)[:, None]


_dis_call = pl.pallas_call(
    _dis_body,
    out_shape=jax.ShapeDtypeStruct((N, 1), jnp.float32),
)


def _tc1_body(x_ref, Win_ref, bin_ref, Wg_ref, dis_ref, h_ref, g_ref):
    h = jnp.dot(x_ref[...], Win_ref[...], preferred_element_type=jnp.float32) + bin_ref[...]
    g = jnp.dot(h, Wg_ref[...], preferred_element_type=jnp.float32) * dis_ref[...]
    h_ref[...] = h
    g_ref[...] = g


_tc1_call = pl.pallas_call(
    _tc1_body,
    grid=(G,),
    in_specs=[
        pl.BlockSpec((R, D), lambda b: (b, 0)),
        pl.BlockSpec((D, D), lambda b: (0, 0)),
        pl.BlockSpec((1, D), lambda b: (0, 0)),
        pl.BlockSpec((D, D), lambda b: (0, 0)),
        pl.BlockSpec((R, 1), lambda b: (b, 0)),
    ],
    out_specs=[
        pl.BlockSpec((R, D), lambda b: (b, 0)),
        pl.BlockSpec((R, D), lambda b: (b, 0)),
    ],
    out_shape=[
        jax.ShapeDtypeStruct((N, D), jnp.float32),
        jax.ShapeDtypeStruct((N, D), jnp.float32),
    ],
)


def _mix(parts, g, h, dis, bg, lnw, lnb):
    acc = parts[0] + parts[1]
    y = dis * (acc + g) + bg
    mu = jnp.mean(y, axis=-1, keepdims=True)
    var = jnp.mean((y - mu) ** 2, axis=-1, keepdims=True)
    z = (y - mu) * lax.rsqrt(var + 1e-5) * lnw + lnb
    tilde = (CVAL - BETA) * jnp.maximum(z, 0.0) + BETA * y
    return tilde + (CVAL - BETA) * h


def _tc2_body(parts_ref, g_ref, h_ref, dis_ref, bg_ref, lnw_ref, lnb_ref,
              Wg_ref, h1_ref, g1_ref):
    dis = dis_ref[...]
    h1 = _mix(parts_ref[...], g_ref[...], h_ref[...], dis,
              bg_ref[...], lnw_ref[...], lnb_ref[...])
    h1_ref[...] = h1
    g1_ref[...] = jnp.dot(h1, Wg_ref[...], preferred_element_type=jnp.float32) * dis


_tc2_call = pl.pallas_call(
    _tc2_body,
    grid=(G,),
    in_specs=[
        pl.BlockSpec((2, R, D), lambda b: (0, b, 0)),
        pl.BlockSpec((R, D), lambda b: (b, 0)),
        pl.BlockSpec((R, D), lambda b: (b, 0)),
        pl.BlockSpec((R, 1), lambda b: (b, 0)),
        pl.BlockSpec((1, D), lambda b: (0, 0)),
        pl.BlockSpec((1, D), lambda b: (0, 0)),
        pl.BlockSpec((1, D), lambda b: (0, 0)),
        pl.BlockSpec((D, D), lambda b: (0, 0)),
    ],
    out_specs=[
        pl.BlockSpec((R, D), lambda b: (b, 0)),
        pl.BlockSpec((R, D), lambda b: (b, 0)),
    ],
    out_shape=[
        jax.ShapeDtypeStruct((N, D), jnp.float32),
        jax.ShapeDtypeStruct((N, D), jnp.float32),
    ],
)


def _tc3_body(parts_ref, g_ref, h_ref, dis_ref, bg_ref, lnw_ref, lnb_ref,
              Wout_ref, bout_ref, out_ref):
    h2 = _mix(parts_ref[...], g_ref[...], h_ref[...], dis_ref[...],
              bg_ref[...], lnw_ref[...], lnb_ref[...])
    out_ref[...] = (jnp.dot(h2, Wout_ref[...], preferred_element_type=jnp.float32)
                    + bout_ref[...])


_tc3_call = pl.pallas_call(
    _tc3_body,
    grid=(G,),
    in_specs=[
        pl.BlockSpec((2, R, D), lambda b: (0, b, 0)),
        pl.BlockSpec((R, D), lambda b: (b, 0)),
        pl.BlockSpec((R, D), lambda b: (b, 0)),
        pl.BlockSpec((R, 1), lambda b: (b, 0)),
        pl.BlockSpec((1, D), lambda b: (0, 0)),
        pl.BlockSpec((1, D), lambda b: (0, 0)),
        pl.BlockSpec((1, D), lambda b: (0, 0)),
        pl.BlockSpec((D, D), lambda b: (0, 0)),
        pl.BlockSpec((1, D), lambda b: (0, 0)),
    ],
    out_specs=pl.BlockSpec((R, D), lambda b: (b, 0)),
    out_shape=jax.ShapeDtypeStruct((N, D), jnp.float32),
)


def kernel(x, edge_index, W_in, b_in, Wg0, bg0, lnw0, lnb0, Wg1, bg1, lnw1,
           lnb1, W_out, b_out):
    src = edge_index[0]
    dst = edge_index[1]
    pad = EPAD - E
    pad_src = jnp.arange(pad, dtype=src.dtype) % N
    pad_dst = N + jnp.arange(pad, dtype=dst.dtype) % (ACC_ROWS - N)
    srcp = jnp.concatenate([src, pad_src]).reshape(NW, STEPS, K)
    dstp = jnp.concatenate([dst, pad_dst]).reshape(NW, STEPS, K)
    dst2d = dst.reshape(NW, DPT)

    vec = lambda v: v.reshape(1, D)

    degp = _deg_call(dst2d)
    dis = _dis_call(degp)
    h, g0 = _tc1_call(x, W_in, vec(b_in), Wg0, dis)
    p0 = _scatter_call(g0, srcp, dstp)
    h1, g1 = _tc2_call(p0, g0, h, dis, vec(bg0), vec(lnw0), vec(lnb0), Wg1)
    p1 = _scatter_call(g1, srcp, dstp)
    out = _tc3_call(p1, g1, h1, dis, vec(bg1), vec(lnw1), vec(lnb1), W_out,
                    vec(b_out))
    return out


# speedup vs baseline: 4.5366x; 1.2537x over previous
import functools

import jax
import jax.numpy as jnp
from jax import lax
from jax.experimental import pallas as pl
from jax.experimental.pallas import tpu as pltpu
from jax.experimental.pallas import tpu_sc as plsc

N = 10000
D = 128
E = 320000
NC = 2
NS = 16
NW = NC * NS
K = 128
STEPS = 80
EPAD = NW * STEPS * K
ACC_ROWS = 10112
STRIPE = ACC_ROWS // NS
DPT = E // NW
BETA = 0.5
CVAL = 1.0

_sc_mesh = plsc.VectorSubcoreMesh(core_axis_name="c", subcore_axis_name="s")


def _deg_body(dst_hbm, out_hbm, idx_v, deg_v):
    c = lax.axis_index("c")
    s = lax.axis_index("s")
    w = s * NC + c
    pltpu.sync_copy(dst_hbm.at[w], idx_v)
    zeros16 = jnp.zeros((16,), jnp.float32)

    def zstep(i, _):
        deg_v[pl.ds(i * 16, 16)] = zeros16
        return 0

    lax.fori_loop(0, DPT // 16, zstep, 0)
    ones16 = jnp.ones((16,), jnp.float32)

    def astep(i, _):
        idx = idx_v[pl.ds(i * 16, 16)]
        plsc.addupdate_scatter(deg_v, [idx], ones16)
        return 0

    lax.fori_loop(0, DPT // 16, astep, 0)
    pltpu.sync_copy(deg_v, out_hbm.at[w])


_deg_call = functools.partial(
    pl.kernel,
    out_type=jax.ShapeDtypeStruct((NW, DPT), jnp.float32),
    mesh=_sc_mesh,
    scratch_types=[
        pltpu.VMEM((DPT,), jnp.int32),
        pltpu.VMEM((DPT,), jnp.float32),
    ],
    compiler_params=pltpu.CompilerParams(needs_layout_passes=False),
)(_deg_body)


def _scatter_body(g_hbm, srcp_hbm, dstp_hbm, out_hbm,
                  idxs, idxd, bufs, gsems, ssems, acc):
    c = lax.axis_index("c")
    s = lax.axis_index("s")
    w = s * NC + c
    zeros16 = jnp.zeros((16,), jnp.float32)

    def zstep(r, _):
        for j in range(D // 16):
            bufs[0][r, pl.ds(j * 16, 16)] = zeros16
        return 0

    lax.fori_loop(0, K, zstep, 0)
    off = 0
    while off < STRIPE:
        n = min(K, STRIPE - off)
        pltpu.sync_copy(bufs[0].at[pl.ds(0, n)],
                        acc.at[pl.ds(s * STRIPE + off, n)])
        off += n
    plsc.subcore_barrier()

    HS = STEPS // 2
    H2 = HS // 2

    def _sidx(l):
        return idxs.at[pl.ds(l * K, K)]

    def _didx(l):
        return idxd.at[pl.ds(l * K, K)]

    def _gather(slot, l):
        pltpu.async_copy(g_hbm.at[_sidx(l)], bufs[slot], gsems[slot])

    def _fire(slot, l):
        pltpu.make_async_copy(g_hbm.at[_sidx(l)], bufs[slot], gsems[slot]).wait()
        pltpu.async_copy(bufs[slot], acc.at[_didx(l)], ssems[slot], add=True)

    def _swait(slot, l):
        pltpu.make_async_copy(bufs[slot], acc.at[_didx(l)], ssems[slot]).wait()

    for half in (0, 1):
        base = half * HS * K
        pltpu.sync_copy(srcp_hbm.at[w, pl.ds(base, HS * K)], idxs)
        pltpu.sync_copy(dstp_hbm.at[w, pl.ds(base, HS * K)], idxd)
        _gather(0, 0)

        def step(j, _):
            l0 = 2 * j

            @pl.when(j >= 1)
            def _():
                _swait(1, l0 - 1)

            _gather(1, l0 + 1)
            _fire(0, l0)

            @pl.when(j < H2 - 1)
            def _():
                _swait(0, l0)
                _gather(0, l0 + 2)

            _fire(1, l0 + 1)
            return 0

        lax.fori_loop(0, H2, step, 0)
        _swait(0, HS - 2)
        _swait(1, HS - 1)
    plsc.subcore_barrier()
    pltpu.sync_copy(acc.at[pl.ds(s * STRIPE, STRIPE)],
                    out_hbm.at[c, pl.ds(s * STRIPE, STRIPE)])


_scatter_call = functools.partial(
    pl.kernel,
    out_type=jax.ShapeDtypeStruct((NC, ACC_ROWS, D), jnp.float32),
    mesh=_sc_mesh,
    scratch_types=[
        pltpu.VMEM((STEPS // 2 * K,), jnp.int32),
        pltpu.VMEM((STEPS // 2 * K,), jnp.int32),
        [pltpu.VMEM((K, D), jnp.float32)] * 2,
        [pltpu.SemaphoreType.DMA] * 2,
        [pltpu.SemaphoreType.DMA] * 2,
        pltpu.VMEM_SHARED((ACC_ROWS, D), jnp.float32),
    ],
    compiler_params=pltpu.CompilerParams(needs_layout_passes=False),
)(_scatter_body)


R = 2000
G = N // R


def _dis_body(degp_ref, dis_ref):
    deg = jnp.sum(degp_ref[...], axis=0) + 1.0
    dis_ref[...] = lax.rsqrt(deg)[:, None]


_dis_call = pl.pallas_call(
    _dis_body,
    out_shape=jax.ShapeDtypeStruct((N, 1), jnp.float32),
)


def _tc1_body(x_ref, Win_ref, bin_ref, Wg_ref, dis_ref, h_ref, g_ref):
    h = jnp.dot(x_ref[...], Win_ref[...], preferred_element_type=jnp.float32) + bin_ref[...]
    g = jnp.dot(h, Wg_ref[...], preferred_element_type=jnp.float32) * dis_ref[...]
    h_ref[...] = h
    g_ref[...] = g


_tc1_call = pl.pallas_call(
    _tc1_body,
    grid=(G,),
    in_specs=[
        pl.BlockSpec((R, D), lambda b: (b, 0)),
        pl.BlockSpec((D, D), lambda b: (0, 0)),
        pl.BlockSpec((1, D), lambda b: (0, 0)),
        pl.BlockSpec((D, D), lambda b: (0, 0)),
        pl.BlockSpec((R, 1), lambda b: (b, 0)),
    ],
    out_specs=[
        pl.BlockSpec((R, D), lambda b: (b, 0)),
        pl.BlockSpec((R, D), lambda b: (b, 0)),
    ],
    out_shape=[
        jax.ShapeDtypeStruct((N, D), jnp.float32),
        jax.ShapeDtypeStruct((N, D), jnp.float32),
    ],
)


def _mix(parts, g, h, dis, bg, lnw, lnb):
    acc = parts[0] + parts[1]
    y = dis * (acc + g) + bg
    mu = jnp.mean(y, axis=-1, keepdims=True)
    var = jnp.mean((y - mu) ** 2, axis=-1, keepdims=True)
    z = (y - mu) * lax.rsqrt(var + 1e-5) * lnw + lnb
    tilde = (CVAL - BETA) * jnp.maximum(z, 0.0) + BETA * y
    return tilde + (CVAL - BETA) * h


def _tc2_body(parts_ref, g_ref, h_ref, dis_ref, bg_ref, lnw_ref, lnb_ref,
              Wg_ref, h1_ref, g1_ref):
    dis = dis_ref[...]
    h1 = _mix(parts_ref[...], g_ref[...], h_ref[...], dis,
              bg_ref[...], lnw_ref[...], lnb_ref[...])
    h1_ref[...] = h1
    g1_ref[...] = jnp.dot(h1, Wg_ref[...], preferred_element_type=jnp.float32) * dis


_tc2_call = pl.pallas_call(
    _tc2_body,
    grid=(G,),
    in_specs=[
        pl.BlockSpec((2, R, D), lambda b: (0, b, 0)),
        pl.BlockSpec((R, D), lambda b: (b, 0)),
        pl.BlockSpec((R, D), lambda b: (b, 0)),
        pl.BlockSpec((R, 1), lambda b: (b, 0)),
        pl.BlockSpec((1, D), lambda b: (0, 0)),
        pl.BlockSpec((1, D), lambda b: (0, 0)),
        pl.BlockSpec((1, D), lambda b: (0, 0)),
        pl.BlockSpec((D, D), lambda b: (0, 0)),
    ],
    out_specs=[
        pl.BlockSpec((R, D), lambda b: (b, 0)),
        pl.BlockSpec((R, D), lambda b: (b, 0)),
    ],
    out_shape=[
        jax.ShapeDtypeStruct((N, D), jnp.float32),
        jax.ShapeDtypeStruct((N, D), jnp.float32),
    ],
)


def _tc3_body(parts_ref, g_ref, h_ref, dis_ref, bg_ref, lnw_ref, lnb_ref,
              Wout_ref, bout_ref, out_ref):
    h2 = _mix(parts_ref[...], g_ref[...], h_ref[...], dis_ref[...],
              bg_ref[...], lnw_ref[...], lnb_ref[...])
    out_ref[...] = (jnp.dot(h2, Wout_ref[...], preferred_element_type=jnp.float32)
                    + bout_ref[...])


_tc3_call = pl.pallas_call(
    _tc3_body,
    grid=(G,),
    in_specs=[
        pl.BlockSpec((2, R, D), lambda b: (0, b, 0)),
        pl.BlockSpec((R, D), lambda b: (b, 0)),
        pl.BlockSpec((R, D), lambda b: (b, 0)),
        pl.BlockSpec((R, 1), lambda b: (b, 0)),
        pl.BlockSpec((1, D), lambda b: (0, 0)),
        pl.BlockSpec((1, D), lambda b: (0, 0)),
        pl.BlockSpec((1, D), lambda b: (0, 0)),
        pl.BlockSpec((D, D), lambda b: (0, 0)),
        pl.BlockSpec((1, D), lambda b: (0, 0)),
    ],
    out_specs=pl.BlockSpec((R, D), lambda b: (b, 0)),
    out_shape=jax.ShapeDtypeStruct((N, D), jnp.float32),
)


def kernel(x, edge_index, W_in, b_in, Wg0, bg0, lnw0, lnb0, Wg1, bg1, lnw1,
           lnb1, W_out, b_out):
    src = edge_index[0]
    dst = edge_index[1]
    pad = EPAD - E
    pad_src = jnp.arange(pad, dtype=src.dtype) % N
    pad_dst = N + jnp.arange(pad, dtype=dst.dtype) % (ACC_ROWS - N)
    srcp = jnp.concatenate([src, pad_src]).reshape(NW, STEPS * K)
    dstp = jnp.concatenate([dst, pad_dst]).reshape(NW, STEPS * K)
    dst2d = dst.reshape(NW, DPT)

    vec = lambda v: v.reshape(1, D)

    degp = _deg_call(dst2d)
    dis = _dis_call(degp)
    h, g0 = _tc1_call(x, W_in, vec(b_in), Wg0, dis)
    p0 = _scatter_call(g0, srcp, dstp)
    h1, g1 = _tc2_call(p0, g0, h, dis, vec(bg0), vec(lnw0), vec(lnb0), Wg1)
    p1 = _scatter_call(g1, srcp, dstp)
    out = _tc3_call(p1, g1, h1, dis, vec(bg1), vec(lnw1), vec(lnb1), W_out,
                    vec(b_out))
    return out


# speedup vs baseline: 4.6237x; 1.0192x over previous
import functools

import jax
import jax.numpy as jnp
from jax import lax
from jax.experimental import pallas as pl
from jax.experimental.pallas import tpu as pltpu
from jax.experimental.pallas import tpu_sc as plsc

N = 10000
D = 128
E = 320000
NC = 2
NS = 16
NW = NC * NS
K = 128
STEPS = 80
EPAD = NW * STEPS * K
ACC_ROWS = 10112
STRIPE = ACC_ROWS // NS
DPT = E // NW
BETA = 0.5
CVAL = 1.0

_sc_mesh = plsc.VectorSubcoreMesh(core_axis_name="c", subcore_axis_name="s")


def _deg_body(dst_hbm, out_hbm, idx_v, deg_v):
    c = lax.axis_index("c")
    s = lax.axis_index("s")
    w = s * NC + c
    pltpu.sync_copy(dst_hbm.at[w], idx_v)
    zeros16 = jnp.zeros((16,), jnp.float32)

    def zstep(i, _):
        deg_v[pl.ds(i * 16, 16)] = zeros16
        return 0

    lax.fori_loop(0, DPT // 16, zstep, 0)
    ones16 = jnp.ones((16,), jnp.float32)

    def astep(i, _):
        idx = idx_v[pl.ds(i * 16, 16)]
        plsc.addupdate_scatter(deg_v, [idx], ones16)
        return 0

    lax.fori_loop(0, DPT // 16, astep, 0)
    pltpu.sync_copy(deg_v, out_hbm.at[w])


_deg_call = functools.partial(
    pl.kernel,
    out_type=jax.ShapeDtypeStruct((NW, DPT), jnp.float32),
    mesh=_sc_mesh,
    scratch_types=[
        pltpu.VMEM((DPT,), jnp.int32),
        pltpu.VMEM((DPT,), jnp.float32),
    ],
    compiler_params=pltpu.CompilerParams(needs_layout_passes=False),
)(_deg_body)


def _scatter_body(g_hbm, srcp_hbm, dstp_hbm, out_hbm,
                  idxs, idxd, bufs, gsems, ssems, acc):
    c = lax.axis_index("c")
    s = lax.axis_index("s")
    w = s * NC + c
    zeros16 = jnp.zeros((16,), jnp.float32)

    def zstep(r, _):
        for j in range(D // 16):
            bufs[0][r, pl.ds(j * 16, 16)] = zeros16
        return 0

    lax.fori_loop(0, K, zstep, 0)
    off = 0
    while off < STRIPE:
        n = min(K, STRIPE - off)
        pltpu.sync_copy(bufs[0].at[pl.ds(0, n)],
                        acc.at[pl.ds(s * STRIPE + off, n)])
        off += n
    plsc.subcore_barrier()

    HS = STEPS // 2
    H2 = HS // 2

    def _sidx(l):
        return idxs.at[pl.ds(l * K, K)]

    def _didx(l):
        return idxd.at[pl.ds(l * K, K)]

    def _gather(slot, l):
        pltpu.async_copy(g_hbm.at[_sidx(l)], bufs[slot], gsems[slot])

    def _fire(slot, l):
        pltpu.make_async_copy(g_hbm.at[_sidx(l)], bufs[slot], gsems[slot]).wait()
        pltpu.async_copy(bufs[slot], acc.at[_didx(l)], ssems[slot], add=True)

    def _swait(slot, l):
        pltpu.make_async_copy(bufs[slot], acc.at[_didx(l)], ssems[slot]).wait()

    for half in (0, 1):
        base = half * HS * K
        pltpu.sync_copy(srcp_hbm.at[w, pl.ds(base, HS * K)], idxs)
        pltpu.sync_copy(dstp_hbm.at[w, pl.ds(base, HS * K)], idxd)
        _gather(0, 0)

        def step(j, _):
            l0 = 2 * j

            @pl.when(j >= 1)
            def _():
                _swait(1, l0 - 1)

            _gather(1, l0 + 1)
            _fire(0, l0)

            @pl.when(j < H2 - 1)
            def _():
                _swait(0, l0)
                _gather(0, l0 + 2)

            _fire(1, l0 + 1)
            return 0

        lax.fori_loop(0, H2, step, 0)
        _swait(0, HS - 2)
        _swait(1, HS - 1)
    plsc.subcore_barrier()
    pltpu.sync_copy(acc.at[pl.ds(s * STRIPE, STRIPE)],
                    out_hbm.at[c, pl.ds(s * STRIPE, STRIPE)])


_scatter_call = functools.partial(
    pl.kernel,
    out_type=jax.ShapeDtypeStruct((NC, ACC_ROWS, D), jnp.float32),
    mesh=_sc_mesh,
    scratch_types=[
        pltpu.VMEM((STEPS // 2 * K,), jnp.int32),
        pltpu.VMEM((STEPS // 2 * K,), jnp.int32),
        [pltpu.VMEM((K, D), jnp.float32)] * 2,
        [pltpu.SemaphoreType.DMA] * 2,
        [pltpu.SemaphoreType.DMA] * 2,
        pltpu.VMEM_SHARED((ACC_ROWS, D), jnp.float32),
    ],
    compiler_params=pltpu.CompilerParams(needs_layout_passes=False),
)(_scatter_body)


R = 2000
G = N // R


def _tc1_body(x_ref, Win_ref, bin_ref, Wg_ref, degp_ref, h_ref, g_ref, dis_ref):
    deg = jnp.sum(degp_ref[...], axis=0) + 1.0
    dis = lax.rsqrt(deg)[:, None]
    h = jnp.dot(x_ref[...], Win_ref[...], preferred_element_type=jnp.float32) + bin_ref[...]
    g = jnp.dot(h, Wg_ref[...], preferred_element_type=jnp.float32) * dis
    h_ref[...] = h
    g_ref[...] = g
    dis_ref[...] = dis


_tc1_call = pl.pallas_call(
    _tc1_body,
    out_shape=[
        jax.ShapeDtypeStruct((N, D), jnp.float32),
        jax.ShapeDtypeStruct((N, D), jnp.float32),
        jax.ShapeDtypeStruct((N, 1), jnp.float32),
    ],
)


def _mix(parts, g, h, dis, bg, lnw, lnb):
    acc = parts[0] + parts[1]
    y = dis * (acc + g) + bg
    mu = jnp.mean(y, axis=-1, keepdims=True)
    var = jnp.mean((y - mu) ** 2, axis=-1, keepdims=True)
    z = (y - mu) * lax.rsqrt(var + 1e-5) * lnw + lnb
    tilde = (CVAL - BETA) * jnp.maximum(z, 0.0) + BETA * y
    return tilde + (CVAL - BETA) * h


def _tc2_body(parts_ref, g_ref, h_ref, dis_ref, bg_ref, lnw_ref, lnb_ref,
              Wg_ref, h1_ref, g1_ref):
    dis = dis_ref[...]
    h1 = _mix(parts_ref[...], g_ref[...], h_ref[...], dis,
              bg_ref[...], lnw_ref[...], lnb_ref[...])
    h1_ref[...] = h1
    g1_ref[...] = jnp.dot(h1, Wg_ref[...], preferred_element_type=jnp.float32) * dis


_tc2_call = pl.pallas_call(
    _tc2_body,
    grid=(G,),
    in_specs=[
        pl.BlockSpec((2, R, D), lambda b: (0, b, 0)),
        pl.BlockSpec((R, D), lambda b: (b, 0)),
        pl.BlockSpec((R, D), lambda b: (b, 0)),
        pl.BlockSpec((R, 1), lambda b: (b, 0)),
        pl.BlockSpec((1, D), lambda b: (0, 0)),
        pl.BlockSpec((1, D), lambda b: (0, 0)),
        pl.BlockSpec((1, D), lambda b: (0, 0)),
        pl.BlockSpec((D, D), lambda b: (0, 0)),
    ],
    out_specs=[
        pl.BlockSpec((R, D), lambda b: (b, 0)),
        pl.BlockSpec((R, D), lambda b: (b, 0)),
    ],
    out_shape=[
        jax.ShapeDtypeStruct((N, D), jnp.float32),
        jax.ShapeDtypeStruct((N, D), jnp.float32),
    ],
)


def _tc3_body(parts_ref, g_ref, h_ref, dis_ref, bg_ref, lnw_ref, lnb_ref,
              Wout_ref, bout_ref, out_ref):
    h2 = _mix(parts_ref[...], g_ref[...], h_ref[...], dis_ref[...],
              bg_ref[...], lnw_ref[...], lnb_ref[...])
    out_ref[...] = (jnp.dot(h2, Wout_ref[...], preferred_element_type=jnp.float32)
                    + bout_ref[...])


_tc3_call = pl.pallas_call(
    _tc3_body,
    grid=(G,),
    in_specs=[
        pl.BlockSpec((2, R, D), lambda b: (0, b, 0)),
        pl.BlockSpec((R, D), lambda b: (b, 0)),
        pl.BlockSpec((R, D), lambda b: (b, 0)),
        pl.BlockSpec((R, 1), lambda b: (b, 0)),
        pl.BlockSpec((1, D), lambda b: (0, 0)),
        pl.BlockSpec((1, D), lambda b: (0, 0)),
        pl.BlockSpec((1, D), lambda b: (0, 0)),
        pl.BlockSpec((D, D), lambda b: (0, 0)),
        pl.BlockSpec((1, D), lambda b: (0, 0)),
    ],
    out_specs=pl.BlockSpec((R, D), lambda b: (b, 0)),
    out_shape=jax.ShapeDtypeStruct((N, D), jnp.float32),
)


def kernel(x, edge_index, W_in, b_in, Wg0, bg0, lnw0, lnb0, Wg1, bg1, lnw1,
           lnb1, W_out, b_out):
    src = edge_index[0]
    dst = edge_index[1]
    pad = EPAD - E
    pad_src = jnp.arange(pad, dtype=src.dtype) % N
    pad_dst = N + jnp.arange(pad, dtype=dst.dtype) % (ACC_ROWS - N)
    srcp = jnp.concatenate([src, pad_src]).reshape(NW, STEPS * K)
    dstp = jnp.concatenate([dst, pad_dst]).reshape(NW, STEPS * K)
    dst2d = dst.reshape(NW, DPT)

    vec = lambda v: v.reshape(1, D)

    degp = _deg_call(dst2d)
    h, g0, dis = _tc1_call(x, W_in, vec(b_in), Wg0, degp)
    p0 = _scatter_call(g0, srcp, dstp)
    h1, g1 = _tc2_call(p0, g0, h, dis, vec(bg0), vec(lnw0), vec(lnb0), Wg1)
    p1 = _scatter_call(g1, srcp, dstp)
    out = _tc3_call(p1, g1, h1, dis, vec(bg1), vec(lnw1), vec(lnb1), W_out,
                    vec(b_out))
    return out
